# baseline (speedup 1.0000x reference)
import dataclasses
import functools

import jax
import jax.numpy as jnp
from jax import lax
from jax.experimental import pallas as pl
from jax.experimental.pallas import tpu as pltpu
from jax.experimental.pallas import tpu_sc as plsc

B = 16384
F = 64
NC = 2
NS = 16
NW = NC * NS
BPW = B // NW
L = 16


def _compiler_params():
    cp = pltpu.CompilerParams(use_tc_tiling_on_sc=False)
    if "needs_layout_passes" in pltpu.CompilerParams.__dataclass_fields__:
        cp = dataclasses.replace(cp, needs_layout_passes=False)
    return cp


def kernel(cell_indices, drug_indices, cell_factors, drug_factors,
           cell_bias, drug_bias, global_bias):
    cell_bias_flat = cell_bias.reshape(-1)
    drug_bias_flat = drug_bias.reshape(-1)
    global_bias16 = jnp.broadcast_to(global_bias, (L,))
    mesh = plsc.VectorSubcoreMesh(core_axis_name="c", subcore_axis_name="s")

    @functools.partial(
        pl.kernel,
        out_type=jax.ShapeDtypeStruct((B,), jnp.float32),
        mesh=mesh,
        compiler_params=_compiler_params(),
        scratch_types=[
            pltpu.VMEM((BPW,), jnp.int32),
            pltpu.VMEM((BPW,), jnp.int32),
            pltpu.VMEM((BPW, F), jnp.float32),
            pltpu.VMEM((BPW, F), jnp.float32),
            pltpu.VMEM((BPW,), jnp.float32),
            pltpu.VMEM((BPW,), jnp.float32),
            pltpu.VMEM((BPW,), jnp.float32),
            pltpu.VMEM((L,), jnp.float32),
            pltpu.SemaphoreType.DMA,
            pltpu.SemaphoreType.DMA,
            pltpu.SemaphoreType.DMA,
            pltpu.SemaphoreType.DMA,
        ],
    )
    def sc_kernel(ci_hbm, di_hbm, cf_hbm, df_hbm, cb_hbm, db_hbm, gb_hbm,
                  out_hbm, ci_v, di_v, cr_v, dr_v, cb_v, db_v, out_v, gb_v,
                  sem0, sem1, sem2, sem3):
        wid = lax.axis_index("s") * NC + lax.axis_index("c")
        base = wid * BPW

        pltpu.sync_copy(ci_hbm.at[pl.ds(base, BPW)], ci_v)
        pltpu.sync_copy(di_hbm.at[pl.ds(base, BPW)], di_v)
        pltpu.sync_copy(gb_hbm, gb_v)

        c_rows = pltpu.async_copy(cf_hbm.at[ci_v], cr_v, sem0)
        c_drows = pltpu.async_copy(df_hbm.at[di_v], dr_v, sem1)
        c_cb = pltpu.async_copy(cb_hbm.at[ci_v], cb_v, sem2)
        c_db = pltpu.async_copy(db_hbm.at[di_v], db_v, sem3)
        c_rows.wait()
        c_drows.wait()
        c_cb.wait()
        c_db.wait()

        g = gb_v[...]
        lanes = lax.iota(jnp.int32, L)

        @pl.loop(0, BPW, step=L)
        def _(r0):
            rows = r0 + lanes
            acc = cb_v[pl.ds(r0, L)] + db_v[pl.ds(r0, L)] + g
            for f in range(F):
                cols = jnp.full((L,), f, jnp.int32)
                cg = plsc.load_gather(cr_v, [rows, cols])
                dg = plsc.load_gather(dr_v, [rows, cols])
                acc = acc + cg * dg
            out_v[pl.ds(r0, L)] = acc

        pltpu.sync_copy(out_v, out_hbm.at[pl.ds(base, BPW)])

    return sc_kernel(cell_indices, drug_indices, cell_factors, drug_factors,
                     cell_bias_flat, drug_bias_flat, global_bias16)
